# baseline (speedup 1.0000x reference)
import functools

import jax
import jax.numpy as jnp
from jax import lax
from jax.experimental import pallas as pl
from jax.experimental.pallas import tpu as pltpu
from jax.experimental.pallas import tpu_sc as plsc

J, M, S, D = 1024, 128, 128, 128
_EPS = 1e-8

_NC, _NS = 2, 16
_NW = _NC * _NS
_NH = _NW // 2
_SM_PER_W = M // _NH
_JB = 128
_JA_PER_W = _JB // _NH
_JREST = J - _JB
_JCHUNK = 32
_NCHUNK = _JREST // _JCHUNK


def _wid():
    return lax.axis_index("s") * _NC + lax.axis_index("c")


def _gather_a_body(jidx_hbm, midx_hbm, sidx_hbm, jtab_hbm, mtab_hbm,
                   stab_hbm, jout_hbm, mout_hbm, sout_hbm,
                   jidx_v, jrows_v, midx_v, mrows_v, sidx_v, srows_v,
                   sem_a, sem_b, sem_c):
    wid = _wid()
    hid = lax.rem(wid, _NH)
    sb = pl.multiple_of(hid * _SM_PER_W, 8)
    jb = pl.multiple_of(hid * _JA_PER_W, 8)

    c1 = pltpu.async_copy(jidx_hbm.at[hid], jidx_v, sem_a)
    c2 = pltpu.async_copy(midx_hbm.at[hid], midx_v, sem_b)
    c3 = pltpu.async_copy(sidx_hbm.at[hid], sidx_v, sem_c)
    c1.wait()
    c2.wait()
    c3.wait()
    g1 = pltpu.async_copy(jtab_hbm.at[jidx_v], jrows_v, sem_a)
    g2 = pltpu.async_copy(mtab_hbm.at[midx_v], mrows_v, sem_b)
    g3 = pltpu.async_copy(stab_hbm.at[sidx_v], srows_v, sem_c)
    g1.wait()
    g2.wait()
    g3.wait()
    w1 = pltpu.async_copy(jrows_v, jout_hbm.at[pl.ds(jb, _JA_PER_W)], sem_a)
    w2 = pltpu.async_copy(mrows_v, mout_hbm.at[pl.ds(sb, _SM_PER_W)], sem_b)
    w3 = pltpu.async_copy(srows_v, sout_hbm.at[pl.ds(sb, _SM_PER_W)], sem_c)
    w1.wait()
    w2.wait()
    w3.wait()


def _gather_b_body(jidx_hbm, jtab_hbm, jout_hbm, jidx_v, jrows_v, sem_a):
    cid = lax.rem(_wid(), _NCHUNK)
    cb = pl.multiple_of(cid * _JCHUNK, 8)
    pltpu.async_copy(jidx_hbm.at[cid], jidx_v, sem_a).wait()
    pltpu.async_copy(jtab_hbm.at[jidx_v], jrows_v, sem_a).wait()
    pltpu.async_copy(jrows_v, jout_hbm.at[pl.ds(cb, _JCHUNK)],
                     sem_a).wait()


@functools.cache
def _gather_a():
    return pl.kernel(
        _gather_a_body,
        mesh=plsc.VectorSubcoreMesh(core_axis_name="c", subcore_axis_name="s"),
        out_type=[
            jax.ShapeDtypeStruct((_JB, D), jnp.float32),
            jax.ShapeDtypeStruct((M, D), jnp.float32),
            jax.ShapeDtypeStruct((S, D), jnp.float32),
        ],
        scratch_types=[
            pltpu.VMEM((_JA_PER_W,), jnp.int32),
            pltpu.VMEM((_JA_PER_W, D), jnp.float32),
            pltpu.VMEM((_SM_PER_W,), jnp.int32),
            pltpu.VMEM((_SM_PER_W, D), jnp.float32),
            pltpu.VMEM((_SM_PER_W,), jnp.int32),
            pltpu.VMEM((_SM_PER_W, D), jnp.float32),
            pltpu.SemaphoreType.DMA,
            pltpu.SemaphoreType.DMA,
            pltpu.SemaphoreType.DMA,
        ],
    )


@functools.cache
def _gather_b():
    return pl.kernel(
        _gather_b_body,
        mesh=plsc.VectorSubcoreMesh(core_axis_name="c", subcore_axis_name="s"),
        out_type=jax.ShapeDtypeStruct((_JREST, D), jnp.float32),
        scratch_types=[
            pltpu.VMEM((_JCHUNK,), jnp.int32),
            pltpu.VMEM((_JCHUNK, D), jnp.float32),
            pltpu.SemaphoreType.DMA,
        ],
    )


def _cosine_outer(je, me, se):
    jn = jnp.sqrt(jnp.sum(je * je, axis=1))
    mn = jnp.sqrt(jnp.sum(me * me, axis=1))
    sn = jnp.sqrt(jnp.sum(se * se, axis=1))
    jm_dot = lax.dot_general(je, me, (((1,), (1,)), ((), ())),
                             preferred_element_type=jnp.float32)
    jm = jm_dot / jnp.maximum(jn[:, None] * mn[None, :], _EPS)
    ms_dot = lax.dot_general(me, se, (((1,), (1,)), ((), ())),
                             preferred_element_type=jnp.float32)
    ms = ms_dot / jnp.maximum(mn[:, None] * sn[None, :], _EPS)
    return jm[:, :, None] * ms[None, :, :]


def _sim1_body(jemb_ref, memb_ref, semb_ref, out_ref):
    out_ref[...] = _cosine_outer(jemb_ref[...], memb_ref[...], semb_ref[...])


def _sim2_body(jemb_ref, memb_ref, semb_ref, prev_ref, out_ref):
    del prev_ref
    out_ref[...] = _cosine_outer(jemb_ref[...], memb_ref[...], semb_ref[...])


def kernel(job_indices, major_indices, subject_indices,
           job_table, major_table, subject_table):
    job_indices = job_indices.astype(jnp.int32)
    jemb_a, memb, semb = _gather_a()(
        job_indices[:_JB].reshape(_NH, _JA_PER_W),
        major_indices.astype(jnp.int32).reshape(_NH, _SM_PER_W),
        subject_indices.astype(jnp.int32).reshape(_NH, _SM_PER_W),
        job_table, major_table, subject_table)
    jemb_b = _gather_b()(
        job_indices[_JB:].reshape(_NCHUNK, _JCHUNK), job_table)
    out1 = pl.pallas_call(
        _sim1_body,
        grid=(1,),
        in_specs=[
            pl.BlockSpec((_JB, D), lambda i: (0, 0)),
            pl.BlockSpec((M, D), lambda i: (0, 0)),
            pl.BlockSpec((S, D), lambda i: (0, 0)),
        ],
        out_specs=pl.BlockSpec((_JB, M, S), lambda i: (0, 0, 0)),
        out_shape=jax.ShapeDtypeStruct((J, M, S), jnp.float32),
    )(jemb_a, memb, semb)
    out = pl.pallas_call(
        _sim2_body,
        grid=(J // _JB - 1,),
        in_specs=[
            pl.BlockSpec((_JB, D), lambda i: (i, 0)),
            pl.BlockSpec((M, D), lambda i: (0, 0)),
            pl.BlockSpec((S, D), lambda i: (0, 0)),
            pl.BlockSpec(memory_space=pl.ANY),
        ],
        out_specs=pl.BlockSpec((_JB, M, S), lambda i: (i + 1, 0, 0)),
        out_shape=jax.ShapeDtypeStruct((J, M, S), jnp.float32),
        input_output_aliases={3: 0},
    )(jemb_b, memb, semb, out1)
    return out.reshape(-1)
